# baseline (speedup 1.0000x reference)
import functools

import jax
import jax.numpy as jnp
from jax import lax
from jax.experimental import pallas as pl
from jax.experimental.pallas import tpu as pltpu
from jax.experimental.pallas import tpu_sc as plsc

_LANES = 16
_NW = 32


def _sc_gather_cur_prev(flat, pos, batch, h):
    bpw = batch // _NW
    mesh = plsc.VectorSubcoreMesh(core_axis_name="c", subcore_axis_name="s")

    @functools.partial(
        pl.kernel,
        mesh=mesh,
        out_type=(
            jax.ShapeDtypeStruct((batch, h), jnp.float32),
            jax.ShapeDtypeStruct((batch, h), jnp.float32),
        ),
        scratch_types=[
            pltpu.VMEM((bpw,), jnp.int32),
            pltpu.VMEM((_LANES,), jnp.int32),
            pltpu.VMEM((_LANES,), jnp.int32),
            pltpu.VMEM((_LANES,), jnp.int32),
            pltpu.VMEM((_LANES,), jnp.int32),
            pltpu.VMEM((2 * bpw, h), jnp.float32),
            pltpu.SemaphoreType.DMA,
            pltpu.SemaphoreType.DMA,
            pltpu.SemaphoreType.DMA,
            pltpu.SemaphoreType.DMA,
            pltpu.SemaphoreType.DMA,
            pltpu.SemaphoreType.DMA,
            pltpu.SemaphoreType.DMA,
            pltpu.SemaphoreType.DMA,
        ],
    )
    def gather_k(flat_hbm, pos_hbm, cur_out, prev_out,
                 pos_v, i0, i1, i2, i3, rows_v, *sems):
        wid = lax.axis_index("s") * 2 + lax.axis_index("c")
        base = wid * bpw
        pltpu.sync_copy(pos_hbm.at[pl.ds(base, bpw)], pos_v)
        idx_refs = (i0, i1, i2, i3)
        for j in range(2):
            p = pos_v[pl.ds(j * _LANES, _LANES)]
            lane = lax.iota(jnp.int32, _LANES) + (base + j * _LANES)
            cur_idx = p * batch + lane
            idx_refs[2 * j][...] = cur_idx
            idx_refs[2 * j + 1][...] = cur_idx - batch
        gathers = []
        for k in range(4):
            gathers.append(pltpu.async_copy(
                flat_hbm.at[idx_refs[k]],
                rows_v.at[pl.ds(k * _LANES, _LANES)], sems[k]))
        writes = []
        for k in range(4):
            j, is_prev = divmod(k, 2)
            dst = prev_out if is_prev else cur_out
            gathers[k].wait()
            writes.append(pltpu.async_copy(
                rows_v.at[pl.ds(k * _LANES, _LANES)],
                dst.at[pl.ds(base + j * _LANES, _LANES)], sems[4 + k]))
        for wdma in writes:
            wdma.wait()

    return gather_k(flat, pos)


def _tc_compose(cur, prev, x, W, b2, op2, dir2):
    batch, h = x.shape
    bb = 256
    dn = (((1,), (1,)), ((), ()))

    def body(cur_ref, prev_ref, x_ref, w_ref, b_ref, op_ref, dir_ref,
             hid_ref, red_ref):
        cur_v = cur_ref[...]
        prev_v = prev_ref[...]
        w = w_ref[...]
        bvec = b_ref[...]
        cc_l = jnp.concatenate([cur_v, prev_v], axis=1)
        cc_r = jnp.concatenate([prev_v, cur_v], axis=1)
        left = jnp.tanh(
            lax.dot_general(cc_l, w, dn, preferred_element_type=jnp.float32) + bvec)
        right = jnp.tanh(
            lax.dot_general(cc_r, w, dn, preferred_element_type=jnp.float32) + bvec)
        opv = op_ref[...]
        drv = dir_ref[...]
        is_left = (opv == -1) & (drv == 0)
        is_right = (opv == -1) & (drv == 1)
        zero = jnp.zeros_like(left)
        red_ref[...] = jnp.where(is_left, left, jnp.where(is_right, right, zero))
        comp = jnp.where(drv == 0, left, right)
        hid_ref[...] = jnp.where(opv == 1, x_ref[...], jnp.where(opv == 0, cur_v, comp))

    return pl.pallas_call(
        body,
        grid=(batch // bb,),
        in_specs=[
            pl.BlockSpec((bb, h), lambda i: (i, 0)),
            pl.BlockSpec((bb, h), lambda i: (i, 0)),
            pl.BlockSpec((bb, h), lambda i: (i, 0)),
            pl.BlockSpec((h, 2 * h), lambda i: (0, 0)),
            pl.BlockSpec((1, h), lambda i: (0, 0)),
            pl.BlockSpec((bb, 1), lambda i: (i, 0)),
            pl.BlockSpec((bb, 1), lambda i: (i, 0)),
        ],
        out_specs=[
            pl.BlockSpec((bb, h), lambda i: (i, 0)),
            pl.BlockSpec((bb, h), lambda i: (i, 0)),
        ],
        out_shape=[
            jax.ShapeDtypeStruct((batch, h), jnp.float32),
            jax.ShapeDtypeStruct((batch, h), jnp.float32),
        ],
    )(cur, prev, x, W, b2, op2, dir2)


def kernel(hidden_stack, x, pos, op, dir_, W, b):
    seq2, batch, h = hidden_stack.shape
    flat = hidden_stack.reshape(seq2 * batch, h)
    pos32 = pos.astype(jnp.int32)
    cur, prev = _sc_gather_cur_prev(flat, pos32, batch, h)
    op2 = op.astype(jnp.int32).reshape(batch, 1)
    dir2 = dir_.astype(jnp.int32).reshape(batch, 1)
    hid, red = _tc_compose(cur, prev, x, W, b.reshape(1, h), op2, dir2)
    return hid, red


# speedup vs baseline: 1.0359x; 1.0359x over previous
import functools

import jax
import jax.numpy as jnp
from jax import lax
from jax.experimental import pallas as pl
from jax.experimental.pallas import tpu as pltpu
from jax.experimental.pallas import tpu_sc as plsc

_LANES = 16
_NW = 32


def _sc_gather_cur_prev(flat, pos, batch, h):
    bpw = batch // _NW
    mesh = plsc.VectorSubcoreMesh(core_axis_name="c", subcore_axis_name="s")

    @functools.partial(
        pl.kernel,
        mesh=mesh,
        out_type=(
            jax.ShapeDtypeStruct((batch, h), jnp.float32),
            jax.ShapeDtypeStruct((batch, h), jnp.float32),
        ),
        scratch_types=[
            pltpu.VMEM((bpw,), jnp.int32),
            pltpu.VMEM((bpw,), jnp.int32),
            pltpu.VMEM((bpw,), jnp.int32),
            pltpu.VMEM((bpw, h), jnp.float32),
            pltpu.VMEM((bpw, h), jnp.float32),
            pltpu.SemaphoreType.DMA,
            pltpu.SemaphoreType.DMA,
            pltpu.SemaphoreType.DMA,
            pltpu.SemaphoreType.DMA,
        ],
    )
    def gather_k(flat_hbm, pos_hbm, cur_out, prev_out,
                 pos_v, idxc_v, idxp_v, rows_c, rows_p, s0, s1, s2, s3):
        wid = lax.axis_index("s") * 2 + lax.axis_index("c")
        base = wid * bpw
        pltpu.sync_copy(pos_hbm.at[pl.ds(base, bpw)], pos_v)
        for j in range(bpw // _LANES):
            p = pos_v[pl.ds(j * _LANES, _LANES)]
            lane = lax.iota(jnp.int32, _LANES) + (base + j * _LANES)
            cur_idx = p * batch + lane
            idxc_v[pl.ds(j * _LANES, _LANES)] = cur_idx
            idxp_v[pl.ds(j * _LANES, _LANES)] = cur_idx - batch
        g_c = pltpu.async_copy(flat_hbm.at[idxc_v], rows_c, s0)
        g_p = pltpu.async_copy(flat_hbm.at[idxp_v], rows_p, s1)
        g_c.wait()
        w_c = pltpu.async_copy(rows_c, cur_out.at[pl.ds(base, bpw)], s2)
        g_p.wait()
        w_p = pltpu.async_copy(rows_p, prev_out.at[pl.ds(base, bpw)], s3)
        w_c.wait()
        w_p.wait()

    return gather_k(flat, pos)


def _tc_compose(cur, prev, x, W, b2, opdir):
    batch, h = x.shape
    bb = 512
    dn = (((1,), (1,)), ((), ()))

    def body(cur_ref, prev_ref, x_ref, w_ref, b_ref, od_ref, hid_ref, red_ref):
        cur_v = cur_ref[...]
        prev_v = prev_ref[...]
        w = w_ref[...]
        bvec = b_ref[...]
        cc_l = jnp.concatenate([cur_v, prev_v], axis=1)
        cc_r = jnp.concatenate([prev_v, cur_v], axis=1)
        left = jnp.tanh(
            lax.dot_general(cc_l, w, dn, preferred_element_type=jnp.float32) + bvec)
        right = jnp.tanh(
            lax.dot_general(cc_r, w, dn, preferred_element_type=jnp.float32) + bvec)
        opv = od_ref[:, 0:1]
        drv = od_ref[:, 1:2]
        is_left = (opv == -1) & (drv == 0)
        is_right = (opv == -1) & (drv == 1)
        zero = jnp.zeros_like(left)
        red_ref[...] = jnp.where(is_left, left, jnp.where(is_right, right, zero))
        comp = jnp.where(drv == 0, left, right)
        hid_ref[...] = jnp.where(opv == 1, x_ref[...], jnp.where(opv == 0, cur_v, comp))

    return pl.pallas_call(
        body,
        grid=(batch // bb,),
        in_specs=[
            pl.BlockSpec((bb, h), lambda i: (i, 0)),
            pl.BlockSpec((bb, h), lambda i: (i, 0)),
            pl.BlockSpec((bb, h), lambda i: (i, 0)),
            pl.BlockSpec((h, 2 * h), lambda i: (0, 0)),
            pl.BlockSpec((1, h), lambda i: (0, 0)),
            pl.BlockSpec((bb, 2), lambda i: (i, 0)),
        ],
        out_specs=[
            pl.BlockSpec((bb, h), lambda i: (i, 0)),
            pl.BlockSpec((bb, h), lambda i: (i, 0)),
        ],
        out_shape=[
            jax.ShapeDtypeStruct((batch, h), jnp.float32),
            jax.ShapeDtypeStruct((batch, h), jnp.float32),
        ],
    )(cur, prev, x, W, b2, opdir)


def kernel(hidden_stack, x, pos, op, dir_, W, b):
    seq2, batch, h = hidden_stack.shape
    flat = hidden_stack.reshape(seq2 * batch, h)
    pos32 = pos.astype(jnp.int32)
    cur, prev = _sc_gather_cur_prev(flat, pos32, batch, h)
    opdir = jnp.stack([op.astype(jnp.int32), dir_.astype(jnp.int32)], axis=1)
    hid, red = _tc_compose(cur, prev, x, W, b.reshape(1, h), opdir)
    return hid, red
